# Initial kernel scaffold; baseline (speedup 1.0000x reference)
import jax
import jax.numpy as jnp
from jax.experimental import pallas as pl


def kernel(inputs, edge_index, W, b):
    raise NotImplementedError("write your pallas kernel here")


# speedup vs baseline: 7.3092x; 7.3092x over previous
import functools

import jax
import jax.numpy as jnp
from jax import lax
from jax.experimental import pallas as pl
from jax.experimental.pallas import tpu as pltpu
from jax.experimental.pallas import tpu_sc as plsc

N_NODES = 10000
N_PAD = 10240
N_EDGES = 320000
D = 128
NC = 2
NS = 16
NW = NC * NS
EPT = N_EDGES // NW
C = 80
NCHUNK = EPT // C
RPT = N_PAD // NS

_mesh = plsc.VectorSubcoreMesh(core_axis_name="c", subcore_axis_name="s")
_f32 = jnp.float32


@functools.partial(
    pl.kernel,
    out_type=jax.ShapeDtypeStruct((NW, N_PAD), _f32),
    mesh=_mesh,
    scratch_types=[
        pltpu.VMEM((EPT,), jnp.int32),
        pltpu.VMEM((N_PAD,), _f32),
    ],
    compiler_params=pltpu.CompilerParams(needs_layout_passes=False),
)
def _deg_kernel(dst_hbm, deg_out, didx, deg_v):
    cid = lax.axis_index("c")
    sid = lax.axis_index("s")
    wid = cid * NS + sid
    pltpu.sync_copy(dst_hbm.at[wid], didx)

    def _zero(i, _):
        deg_v[pl.ds(i * 16, 16)] = jnp.zeros((16,), _f32)
        return 0

    lax.fori_loop(0, N_PAD // 16, _zero, 0)

    ones = jnp.ones((16,), _f32)

    def _acc(i, _):
        idx = didx[pl.ds(i * 16, 16)]
        plsc.addupdate_scatter(deg_v, [idx], ones)
        return 0

    lax.fori_loop(0, EPT // 16, _acc, 0)
    pltpu.sync_copy(deg_v, deg_out.at[wid])


@functools.partial(
    pl.kernel,
    out_type=jax.ShapeDtypeStruct((NC, N_PAD, D), _f32),
    mesh=_mesh,
    scratch_types=[
        pltpu.VMEM((NCHUNK, C), jnp.int32),
        pltpu.VMEM((NCHUNK, C), jnp.int32),
        pltpu.VMEM((C, D), _f32),
        pltpu.SemaphoreType.DMA,
        pltpu.VMEM_SHARED((N_PAD, D), _f32),
    ],
)
def _prop_kernel(tin_hbm, src_hbm, dst_hbm, out_hbm,
                 sidx, didx, rows, sem, acc):
    cid = lax.axis_index("c")
    sid = lax.axis_index("s")
    wid = cid * NS + sid
    pltpu.sync_copy(src_hbm.at[wid], sidx)
    pltpu.sync_copy(dst_hbm.at[wid], didx)

    def _zrow(r, _):
        for cblk in range(D // 16):
            rows[r, pl.ds(cblk * 16, 16)] = jnp.zeros((16,), _f32)
        return 0

    lax.fori_loop(0, C, _zrow, 0)
    for m in range(RPT // C):
        pltpu.sync_copy(rows, acc.at[pl.ds(sid * RPT + m * C, C)])
    plsc.subcore_barrier()

    def _chunk(j, _):
        pltpu.async_copy(tin_hbm.at[sidx.at[j]], rows, sem).wait()
        pltpu.sync_copy(rows, acc.at[didx.at[j]], add=True)
        return 0

    lax.fori_loop(0, NCHUNK, _chunk, 0)
    plsc.subcore_barrier()

    for m in range(RPT // C):
        r0 = sid * RPT + m * C
        pltpu.sync_copy(acc.at[pl.ds(r0, C)], rows)
        pltpu.sync_copy(rows, out_hbm.at[cid, pl.ds(r0, C)])


def _scale_body(x_ref, degp_ref, o_ref):
    deg = jnp.sum(degp_ref[...], axis=1, keepdims=True)
    norm = lax.rsqrt(jnp.maximum(deg, 1.0))
    o_ref[...] = x_ref[...] * norm


def _combine_body(parts_ref, degp_ref, o_ref):
    deg = jnp.sum(degp_ref[...], axis=1, keepdims=True)
    inv = 1.0 / jnp.maximum(deg, 1.0)
    o_ref[...] = (parts_ref[0] + parts_ref[1]) * inv


def _final_body(parts_ref, degp_ref, w_ref, b_ref, o_ref):
    deg = jnp.sum(degp_ref[...], axis=1, keepdims=True)
    norm = lax.rsqrt(jnp.maximum(deg, 1.0))
    s = (parts_ref[0] + parts_ref[1]) * norm
    o_ref[...] = jnp.dot(s, w_ref[...], preferred_element_type=_f32) + b_ref[...]


_ROWS_BLK = 512
_GRID = N_PAD // _ROWS_BLK


def _tc_scale(xp, degp_t):
    return pl.pallas_call(
        _scale_body,
        grid=(_GRID,),
        in_specs=[
            pl.BlockSpec((_ROWS_BLK, D), lambda i: (i, 0)),
            pl.BlockSpec((_ROWS_BLK, NW), lambda i: (i, 0)),
        ],
        out_specs=pl.BlockSpec((_ROWS_BLK, D), lambda i: (i, 0)),
        out_shape=jax.ShapeDtypeStruct((N_PAD, D), _f32),
    )(xp, degp_t)


def _tc_combine(parts, degp_t):
    return pl.pallas_call(
        _combine_body,
        grid=(_GRID,),
        in_specs=[
            pl.BlockSpec((NC, _ROWS_BLK, D), lambda i: (0, i, 0)),
            pl.BlockSpec((_ROWS_BLK, NW), lambda i: (i, 0)),
        ],
        out_specs=pl.BlockSpec((_ROWS_BLK, D), lambda i: (i, 0)),
        out_shape=jax.ShapeDtypeStruct((N_PAD, D), _f32),
    )(parts, degp_t)


def _tc_final(parts, degp_t, W, b2):
    return pl.pallas_call(
        _final_body,
        grid=(_GRID,),
        in_specs=[
            pl.BlockSpec((NC, _ROWS_BLK, D), lambda i: (0, i, 0)),
            pl.BlockSpec((_ROWS_BLK, NW), lambda i: (i, 0)),
            pl.BlockSpec((D, D), lambda i: (0, 0)),
            pl.BlockSpec((1, D), lambda i: (0, 0)),
        ],
        out_specs=pl.BlockSpec((_ROWS_BLK, D), lambda i: (i, 0)),
        out_shape=jax.ShapeDtypeStruct((N_PAD, D), _f32),
    )(parts, degp_t, W, b2)


def kernel(inputs, edge_index, W, b):
    src = edge_index[0].astype(jnp.int32).reshape(NW, NCHUNK, C)
    dst = edge_index[1].astype(jnp.int32).reshape(NW, NCHUNK, C)
    dst_deg = dst.reshape(NW, EPT)
    xp = jnp.pad(inputs, ((0, N_PAD - N_NODES), (0, 0)))
    b2 = b.reshape(1, D)

    degp_t = _deg_kernel(dst_deg).T

    t0 = _tc_scale(xp, degp_t)
    p1 = _prop_kernel(t0, src, dst)
    t1 = _tc_combine(p1, degp_t)
    p2 = _prop_kernel(t1, src, dst)
    h = _tc_final(p2, degp_t, W, b2)

    return (h[:N_NODES], 0)
